# Initial kernel scaffold; baseline (speedup 1.0000x reference)
import jax
import jax.numpy as jnp
from jax.experimental import pallas as pl


def kernel(token_ids, embed_weight, proj_weight, scale):
    raise NotImplementedError("write your pallas kernel here")


# speedup vs baseline: 2.9120x; 2.9120x over previous
import functools

import jax
import jax.numpy as jnp
from jax import lax
from jax.experimental import pallas as pl
from jax.experimental.pallas import tpu as pltpu
from jax.experimental.pallas import tpu_sc as plsc

_BGVS = 1000000
_BGD = 128
_DM = 2048
_B, _S = 4, 4096
_N = _B * _S
_NW = 32
_PER_W = _N // _NW
_GCHUNK = 128
_NCHUNK = _PER_W // _GCHUNK


def _sc_hash_gather(tok_flat, tok_prev, table):
    mesh = plsc.VectorSubcoreMesh(core_axis_name="c", subcore_axis_name="s")

    @functools.partial(
        pl.kernel,
        mesh=mesh,
        out_type=jax.ShapeDtypeStruct((_N, _BGD), jnp.float32),
        scratch_types=[
            pltpu.VMEM((_PER_W,), jnp.int32),
            pltpu.VMEM((_PER_W,), jnp.int32),
            pltpu.VMEM((_NCHUNK, _GCHUNK), jnp.int32),
            pltpu.VMEM((_PER_W, _BGD), jnp.float32),
            pltpu.SemaphoreType.DMA,
        ],
    )
    def k(tok_hbm, prev_hbm, table_hbm, out_hbm, tok_v, prev_v, idx_v, rows_v, sem):
        wid = lax.axis_index("s") * 2 + lax.axis_index("c")
        base = wid * _PER_W
        pltpu.sync_copy(tok_hbm.at[pl.ds(base, _PER_W)], tok_v)
        pltpu.sync_copy(prev_hbm.at[pl.ds(base, _PER_W)], prev_v)
        seq_start = 1 - jnp.minimum(jnp.int32(1), base % _S)
        lane0 = jnp.maximum(jnp.int32(0), 1 - lax.iota(jnp.int32, 16))
        for v in range(_PER_W // 16):
            cur = tok_v[pl.ds(v * 16, 16)]
            prv = prev_v[pl.ds(v * 16, 16)]
            h = jnp.mod(
                jnp.bitwise_xor(jnp.int32(36313) * cur, jnp.int32(27191) * prv),
                jnp.int32(_BGVS - 1),
            )
            if v == 0:
                sel = lane0 * seq_start
                h = h + sel * (jnp.int32(_BGVS - 1) - h)
            idx_v[v // 8, pl.ds((v % 8) * 16, 16)] = h
        copies = [
            pltpu.async_copy(
                table_hbm.at[idx_v.at[c]],
                rows_v.at[pl.ds(c * _GCHUNK, _GCHUNK)],
                sem,
            )
            for c in range(_NCHUNK)
        ]
        for cp in copies:
            cp.wait()
        pltpu.sync_copy(rows_v, out_hbm.at[pl.ds(base, _PER_W)])

    return k(tok_flat, tok_prev, table)


def _tc_matmul(x, w, scale):
    blk = 512

    def mm(scale_ref, x_ref, w_ref, o_ref):
        acc = lax.dot_general(
            x_ref[...], w_ref[...],
            (((1,), (1,)), ((), ())),
            preferred_element_type=jnp.float32,
        )
        o_ref[...] = acc * scale_ref[0]

    return pl.pallas_call(
        mm,
        grid=(_N // blk,),
        in_specs=[
            pl.BlockSpec(memory_space=pltpu.SMEM),
            pl.BlockSpec((blk, _BGD), lambda i: (i, 0)),
            pl.BlockSpec((_DM, _BGD), lambda i: (0, 0)),
        ],
        out_specs=pl.BlockSpec((blk, _DM), lambda i: (i, 0)),
        out_shape=jax.ShapeDtypeStruct((_N, _DM), jnp.float32),
    )(scale.reshape(1), x, w)


def kernel(token_ids, embed_weight, proj_weight, scale):
    flat = token_ids.reshape(-1).astype(jnp.int32)
    prev = jnp.concatenate([jnp.zeros((1,), jnp.int32), flat[:-1]])
    gathered = _sc_hash_gather(flat, prev, embed_weight)
    out = _tc_matmul(gathered, proj_weight, scale)
    return out.reshape(_B, _S, _DM)
